# baseline (speedup 1.0000x reference)
import functools
import math

import jax
import jax.numpy as jnp
from jax import lax
from jax.experimental import pallas as pl
from jax.experimental.pallas import tpu as pltpu
from jax.experimental.pallas import tpu_sc as plsc

DIM = 64
SCALE = 1.0 / math.sqrt(DIM)
LANES = 16
NUM_CORES = 2
NUM_SUBCORES = 16
NW = NUM_CORES * NUM_SUBCORES

B = 4096 * 200
B_PER_W = B // NW
CHUNK = 512
N_CHUNKS = B_PER_W // CHUNK


def _emb_kernel(x_hbm, table_hbm, out_hbm, idx_v, rows_v, sem):
    wid = lax.axis_index("s") * NUM_CORES + lax.axis_index("c")
    base = wid * B_PER_W

    def chunk_body(i, carry):
        off = base + i * CHUNK
        pltpu.sync_copy(x_hbm.at[pl.ds(off, CHUNK)], idx_v)
        pltpu.async_copy(table_hbm.at[idx_v], rows_v, sem).wait()

        def scale_row(r, c):
            for j in range(DIM // LANES):
                sl = pl.ds(j * LANES, LANES)
                rows_v[r, sl] = rows_v[r, sl] * SCALE
            return c

        lax.fori_loop(0, CHUNK, scale_row, 0)
        pltpu.sync_copy(rows_v, out_hbm.at[pl.ds(off, CHUNK)])
        return carry

    lax.fori_loop(0, N_CHUNKS, chunk_body, 0)


@jax.jit
def kernel(x, table):
    x_flat = x.reshape(B).astype(jnp.int32)
    mesh = plsc.VectorSubcoreMesh(core_axis_name="c", subcore_axis_name="s")
    out = pl.kernel(
        _emb_kernel,
        mesh=mesh,
        out_type=jax.ShapeDtypeStruct((B, DIM), jnp.float32),
        scratch_types=[
            pltpu.VMEM((CHUNK,), jnp.int32),
            pltpu.VMEM((CHUNK, DIM), jnp.float32),
            pltpu.SemaphoreType.DMA,
        ],
        compiler_params=pltpu.CompilerParams(use_tc_tiling_on_sc=False),
    )(x_flat, table)
    return out.reshape(x.shape + (DIM,))


# speedup vs baseline: 1.5788x; 1.5788x over previous
import math

import jax
import jax.numpy as jnp
from jax import lax
from jax.experimental import pallas as pl
from jax.experimental.pallas import tpu as pltpu
from jax.experimental.pallas import tpu_sc as plsc

DIM = 64
SCALE = 1.0 / math.sqrt(DIM)
LANES = 16
NUM_CORES = 2
NUM_SUBCORES = 16
VOCAB = 1000000
S = 200
NB = 4096
F_PER_CORE = DIM // NUM_CORES
S_MAX = 13


def _emb_kernel(xT_hbm, tableT_hbm, out_hbm, *refs):
    idx_bufs = refs[:S_MAX]
    gath_v, row_sp, sem = refs[S_MAX:]
    c = lax.axis_index("c")
    t = lax.axis_index("s")
    s0 = jnp.where(t < 8, t * 13, 104 + (t - 8) * 12)

    for k in range(S_MAX - 1):
        pltpu.sync_copy(xT_hbm.at[s0 + k], idx_bufs[k])

    @pl.when(t < 8)
    def _():
        pltpu.sync_copy(xT_hbm.at[s0 + (S_MAX - 1)], idx_bufs[S_MAX - 1])

    def feat_body(j, carry):
        f = c * F_PER_CORE + j

        @pl.when(t == 0)
        def _():
            pltpu.sync_copy(tableT_hbm.at[f], row_sp)

        plsc.subcore_barrier()

        def serve(k):
            pltpu.async_copy(row_sp.at[idx_bufs[k]], gath_v, sem).wait()

            def scale_blk(v, c2):
                for u in range(8):
                    sl = pl.ds((v * 8 + u) * LANES, LANES)
                    gath_v[sl] = gath_v[sl] * SCALE
                return c2

            lax.fori_loop(0, NB // (8 * LANES), scale_blk, 0)
            pltpu.sync_copy(gath_v, out_hbm.at[s0 + k, f])

        for k in range(S_MAX - 1):
            serve(k)

        @pl.when(t < 8)
        def _():
            serve(S_MAX - 1)

        plsc.subcore_barrier()
        return carry

    lax.fori_loop(0, F_PER_CORE, feat_body, 0)


@jax.jit
def kernel(x, table):
    xT = x.T.astype(jnp.int32)
    tableT = table.T
    mesh = plsc.VectorSubcoreMesh(core_axis_name="c", subcore_axis_name="s")
    out_phys = pl.kernel(
        _emb_kernel,
        mesh=mesh,
        out_type=jax.ShapeDtypeStruct((S, DIM, NB), jnp.float32),
        scratch_types=(
            [pltpu.VMEM((NB,), jnp.int32) for _ in range(S_MAX)]
            + [
                pltpu.VMEM((NB,), jnp.float32),
                pltpu.VMEM_SHARED((VOCAB,), jnp.float32),
                pltpu.SemaphoreType.DMA,
            ]
        ),
    )(xT, tableT)
    return out_phys.transpose(2, 0, 1)


# speedup vs baseline: 1.9704x; 1.2480x over previous
import math

import jax
import jax.numpy as jnp
from jax import lax
from jax.experimental import pallas as pl
from jax.experimental.pallas import tpu as pltpu
from jax.experimental.pallas import tpu_sc as plsc

DIM = 64
SCALE = 1.0 / math.sqrt(DIM)
LANES = 16
NUM_CORES = 2
NUM_SUBCORES = 16
VOCAB = 1000000
S = 200
NB = 4096
F_PER_CORE = DIM // NUM_CORES
S_MAX = 13

def _emb_kernel(xT_hbm, tableT_hbm, out_hbm, *refs):
    idx_bufs = refs[:S_MAX]
    dat_bufs = refs[S_MAX:2 * S_MAX]
    row_sp, sem_g, sem_w, sem_s = refs[2 * S_MAX:]
    c = lax.axis_index("c")
    t = lax.axis_index("s")
    s0 = jnp.where(t < 8, t * 13, 104 + (t - 8) * 12)

    for k in range(S_MAX - 1):
        pltpu.sync_copy(xT_hbm.at[s0 + k], idx_bufs[k])

    @pl.when(t < 8)
    def _():
        pltpu.sync_copy(xT_hbm.at[s0 + (S_MAX - 1)], idx_bufs[S_MAX - 1])

    def stage_row(f):
        @pl.when(t == 0)
        def _():
            pltpu.async_copy(tableT_hbm.at[f], row_sp, sem_s)

    def wait_stage():
        @pl.when(t == 0)
        def _():
            pltpu.make_async_copy(tableT_hbm.at[0], row_sp, sem_s).wait()

    def scale_buf(buf):
        def scale_blk(v, c2):
            for u in range(8):
                sl = pl.ds((v * 8 + u) * LANES, LANES)
                buf[sl] = buf[sl] * SCALE
            return c2

        lax.fori_loop(0, NB // (8 * LANES), scale_blk, 0)

    stage_row(c * F_PER_CORE)
    wait_stage()
    plsc.subcore_barrier()

    NRING = 3

    def fire_gather(k):
        pltpu.async_copy(row_sp.at[idx_bufs[k]], dat_bufs[k % NRING], sem_g)

    def wait_gather(k):
        pltpu.make_async_copy(
            row_sp.at[idx_bufs[k]], dat_bufs[k % NRING], sem_g
        ).wait()

    def fire_write(k, f):
        pltpu.async_copy(dat_bufs[k % NRING], out_hbm.at[s0 + k, f], sem_w)

    def wait_write(k):
        pltpu.make_async_copy(
            dat_bufs[k % NRING], out_hbm.at[0, 0], sem_w
        ).wait()

    def feat_body(j, carry):
        f = c * F_PER_CORE + j

        fire_gather(0)
        fire_gather(1)

        for k in range(S_MAX):
            if k == S_MAX - 1:
                @pl.when(t < 8)
                def _():
                    wait_gather(S_MAX - 1)
                    scale_buf(dat_bufs[(S_MAX - 1) % NRING])
                    fire_write(S_MAX - 1, f)
            else:
                wait_gather(k)
                scale_buf(dat_bufs[k % NRING])
                fire_write(k, f)

            nk = k + 2
            if nk == S_MAX - 1:
                @pl.when(t < 8)
                def _():
                    wait_write(k - 1)
                    fire_gather(S_MAX - 1)
            elif nk < S_MAX - 1:
                if k >= 1:
                    wait_write(k - 1)
                fire_gather(nk)

        plsc.subcore_barrier()

        @pl.when(j + 1 < F_PER_CORE)
        def _():
            stage_row(f + 1)

        for k in range(NRING):
            wait_write(k)

        @pl.when(j + 1 < F_PER_CORE)
        def _():
            wait_stage()

        plsc.subcore_barrier()
        return carry

    lax.fori_loop(0, F_PER_CORE, feat_body, 0)


@jax.jit
def kernel(x, table):
    xT = x.T.astype(jnp.int32)
    tableT = table.T
    mesh = plsc.VectorSubcoreMesh(core_axis_name="c", subcore_axis_name="s")
    out_phys = pl.kernel(
        _emb_kernel,
        mesh=mesh,
        out_type=jax.ShapeDtypeStruct((S, DIM, NB), jnp.float32),
        scratch_types=(
            [pltpu.VMEM((NB,), jnp.int32) for _ in range(S_MAX)]
            + [pltpu.VMEM((NB,), jnp.float32) for _ in range(S_MAX)]
            + [
                pltpu.VMEM_SHARED((VOCAB,), jnp.float32),
                pltpu.SemaphoreType.DMA,
                pltpu.SemaphoreType.DMA,
                pltpu.SemaphoreType.DMA,
            ]
        ),
    )(xT, tableT)
    return out_phys.transpose(2, 0, 1)


# speedup vs baseline: 1.9779x; 1.0038x over previous
import math

import jax
import jax.numpy as jnp
from jax import lax
from jax.experimental import pallas as pl
from jax.experimental.pallas import tpu as pltpu
from jax.experimental.pallas import tpu_sc as plsc

DIM = 64
SCALE = 1.0 / math.sqrt(DIM)
LANES = 16
NUM_CORES = 2
NUM_SUBCORES = 16
VOCAB = 1000000
S = 200
NB = 4096
F_PER_CORE = DIM // NUM_CORES
S_MAX = 13

def _emb_kernel(xT_hbm, tableT_hbm, tailT_hbm, out_hbm, *refs):
    idx_bufs = refs[:S_MAX]
    dat_bufs = refs[S_MAX:2 * S_MAX]
    tail_buf, tail_blk, row_sp, sem_g, sem_w, sem_s, sem_t = refs[2 * S_MAX:]
    c = lax.axis_index("c")
    t = lax.axis_index("s")
    s0 = jnp.where(t < 8, t * 13, 104 + (t - 8) * 12)

    for k in range(S_MAX - 1):
        pltpu.sync_copy(xT_hbm.at[s0 + k], idx_bufs[k])

    @pl.when(t < 8)
    def _():
        pltpu.sync_copy(xT_hbm.at[s0 + (S_MAX - 1)], idx_bufs[S_MAX - 1])

    STAGE_CHUNK = 62464
    STAGE_LAST = 62976
    TAIL = 128
    TAIL0 = VOCAB - TAIL

    def stage_row(f):
        row_hbm = tableT_hbm.at[f]

        @pl.when(t < NUM_SUBCORES - 1)
        def _():
            pltpu.async_copy(
                row_hbm.at[pl.ds(t * STAGE_CHUNK, STAGE_CHUNK)],
                row_sp.at[pl.ds(t * STAGE_CHUNK, STAGE_CHUNK)],
                sem_s,
            )

        @pl.when(t == NUM_SUBCORES - 1)
        def _():
            pltpu.async_copy(
                row_hbm.at[pl.ds(15 * STAGE_CHUNK, STAGE_LAST)],
                row_sp.at[pl.ds(15 * STAGE_CHUNK, STAGE_LAST)],
                sem_s,
            )
            pltpu.sync_copy(tail_blk.at[f], tail_buf)
            pltpu.async_copy(tail_buf, row_sp.at[pl.ds(TAIL0, TAIL)], sem_t)

    def wait_stage():
        row_hbm = tableT_hbm.at[0]

        @pl.when(t < NUM_SUBCORES - 1)
        def _():
            pltpu.make_async_copy(
                row_hbm.at[pl.ds(0, STAGE_CHUNK)],
                row_sp.at[pl.ds(0, STAGE_CHUNK)],
                sem_s,
            ).wait()

        @pl.when(t == NUM_SUBCORES - 1)
        def _():
            pltpu.make_async_copy(
                row_hbm.at[pl.ds(0, STAGE_LAST)],
                row_sp.at[pl.ds(0, STAGE_LAST)],
                sem_s,
            ).wait()
            pltpu.make_async_copy(
                tail_buf, row_sp.at[pl.ds(TAIL0, TAIL)], sem_t
            ).wait()

    def scale_buf(buf):
        def scale_blk(v, c2):
            for u in range(8):
                sl = pl.ds((v * 8 + u) * LANES, LANES)
                buf[sl] = buf[sl] * SCALE
            return c2

        lax.fori_loop(0, NB // (8 * LANES), scale_blk, 0)

    @pl.when(t == NUM_SUBCORES - 1)
    def _():
        pltpu.sync_copy(tailT_hbm, tail_blk)

    stage_row(c * F_PER_CORE)
    wait_stage()
    plsc.subcore_barrier()

    NRING = 3

    def fire_gather(k):
        pltpu.async_copy(row_sp.at[idx_bufs[k]], dat_bufs[k % NRING], sem_g)

    def wait_gather(k):
        pltpu.make_async_copy(
            row_sp.at[idx_bufs[k]], dat_bufs[k % NRING], sem_g
        ).wait()

    def fire_write(k, f):
        pltpu.async_copy(dat_bufs[k % NRING], out_hbm.at[s0 + k, f], sem_w)

    def wait_write(k):
        pltpu.make_async_copy(
            dat_bufs[k % NRING], out_hbm.at[0, 0], sem_w
        ).wait()

    def feat_body(j, carry):
        f = c * F_PER_CORE + j

        fire_gather(0)
        fire_gather(1)

        for k in range(S_MAX):
            if k == S_MAX - 1:
                @pl.when(t < 8)
                def _():
                    wait_gather(S_MAX - 1)
                    scale_buf(dat_bufs[(S_MAX - 1) % NRING])
                    fire_write(S_MAX - 1, f)
            else:
                wait_gather(k)
                scale_buf(dat_bufs[k % NRING])
                fire_write(k, f)

            nk = k + 2
            if nk == S_MAX - 1:
                @pl.when(t < 8)
                def _():
                    wait_write(k - 1)
                    fire_gather(S_MAX - 1)
            elif nk < S_MAX - 1:
                if k >= 1:
                    wait_write(k - 1)
                fire_gather(nk)

        plsc.subcore_barrier()

        @pl.when(j + 1 < F_PER_CORE)
        def _():
            stage_row(f + 1)

        for k in range(NRING):
            wait_write(k)

        @pl.when(j + 1 < F_PER_CORE)
        def _():
            wait_stage()

        plsc.subcore_barrier()
        return carry

    lax.fori_loop(0, F_PER_CORE, feat_body, 0)


@jax.jit
def kernel(x, table):
    xT = x.T.astype(jnp.int32)
    tableT = table.T
    tailT = jax.lax.slice(tableT, (0, VOCAB - 128), (DIM, VOCAB))
    mesh = plsc.VectorSubcoreMesh(core_axis_name="c", subcore_axis_name="s")
    out_phys = pl.kernel(
        _emb_kernel,
        mesh=mesh,
        out_type=jax.ShapeDtypeStruct((S, DIM, NB), jnp.float32),
        scratch_types=(
            [pltpu.VMEM((NB,), jnp.int32) for _ in range(S_MAX)]
            + [pltpu.VMEM((NB,), jnp.float32) for _ in range(S_MAX)]
            + [
                pltpu.VMEM((128,), jnp.float32),
                pltpu.VMEM_SHARED((DIM, 128), jnp.float32),
                pltpu.VMEM_SHARED((VOCAB,), jnp.float32),
                pltpu.SemaphoreType.DMA,
                pltpu.SemaphoreType.DMA,
                pltpu.SemaphoreType.DMA,
                pltpu.SemaphoreType.DMA,
            ]
        ),
    )(xT, tableT, tailT)
    return out_phys.transpose(2, 0, 1)


# speedup vs baseline: 1.9841x; 1.0032x over previous
import math

import jax
import jax.numpy as jnp
from jax import lax
from jax.experimental import pallas as pl
from jax.experimental.pallas import tpu as pltpu
from jax.experimental.pallas import tpu_sc as plsc

DIM = 64
SCALE = 1.0 / math.sqrt(DIM)
LANES = 16
NUM_CORES = 2
NUM_SUBCORES = 16
VOCAB = 1000000
S = 200
NB = 4096
F_PER_CORE = DIM // NUM_CORES
S_MAX = 13

def _emb_kernel(xT_hbm, tableT_hbm, out_hbm, *refs):
    idx_bufs = refs[:S_MAX]
    dat_bufs = refs[S_MAX:2 * S_MAX]
    row_sp, sem_g, sem_w, sem_s = refs[2 * S_MAX:]
    c = lax.axis_index("c")
    t = lax.axis_index("s")
    s0 = jnp.where(t < 8, t * 13, 104 + (t - 8) * 12)

    for k in range(S_MAX - 1):
        pltpu.sync_copy(xT_hbm.at[s0 + k], idx_bufs[k])

    @pl.when(t < 8)
    def _():
        pltpu.sync_copy(xT_hbm.at[s0 + (S_MAX - 1)], idx_bufs[S_MAX - 1])

    def stage_row(f):
        @pl.when(t == 0)
        def _():
            pltpu.async_copy(tableT_hbm.at[f], row_sp, sem_s)

    def wait_stage():
        @pl.when(t == 0)
        def _():
            pltpu.make_async_copy(tableT_hbm.at[0], row_sp, sem_s).wait()

    def scale_buf(buf):
        def scale_blk(v, c2):
            for u in range(8):
                sl = pl.ds((v * 8 + u) * LANES, LANES)
                buf[sl] = buf[sl] * SCALE
            return c2

        lax.fori_loop(0, NB // (8 * LANES), scale_blk, 0)

    stage_row(c * F_PER_CORE)
    wait_stage()
    plsc.subcore_barrier()

    NRING = 3

    def fire_gather(k):
        pltpu.async_copy(row_sp.at[idx_bufs[k]], dat_bufs[k % NRING], sem_g)

    def wait_gather(k):
        pltpu.make_async_copy(
            row_sp.at[idx_bufs[k]], dat_bufs[k % NRING], sem_g
        ).wait()

    def fire_write(k, f):
        pltpu.async_copy(dat_bufs[k % NRING], out_hbm.at[s0 + k, f], sem_w)

    def wait_write(k):
        pltpu.make_async_copy(
            dat_bufs[k % NRING], out_hbm.at[0, 0], sem_w
        ).wait()

    def feat_body(j, carry):
        f = c * F_PER_CORE + j

        fire_gather(0)
        fire_gather(1)

        for k in range(S_MAX):
            if k == 9:
                wait_gather(k)

                @pl.when(t < 8)
                def _():
                    scale_buf(dat_bufs[k % NRING])
                    fire_write(k, f)
            elif k in (10, 11):
                wait_gather(k)
            elif k == S_MAX - 1:
                @pl.when(t < 8)
                def _():
                    wait_gather(S_MAX - 1)
            else:
                wait_gather(k)
                scale_buf(dat_bufs[k % NRING])
                fire_write(k, f)

            nk = k + 2
            if nk == S_MAX - 1:
                @pl.when(t < 8)
                def _():
                    wait_write(k - 1)
                    fire_gather(S_MAX - 1)
            elif nk < S_MAX - 1:
                if k >= 1:
                    wait_write(k - 1)
                fire_gather(nk)

        plsc.subcore_barrier()

        @pl.when(j + 1 < F_PER_CORE)
        def _():
            stage_row(f + 1)

        @pl.when(t < 8)
        def _():
            for k in (10, 11, 12):
                scale_buf(dat_bufs[k % NRING])
                fire_write(k, f)

        @pl.when(t >= 8)
        def _():
            for k in (9, 10, 11):
                scale_buf(dat_bufs[k % NRING])
                fire_write(k, f)

        for k in range(NRING):
            wait_write(k)

        @pl.when(j + 1 < F_PER_CORE)
        def _():
            wait_stage()

        plsc.subcore_barrier()
        return carry

    lax.fori_loop(0, F_PER_CORE, feat_body, 0)


@jax.jit
def kernel(x, table):
    xT = x.T.astype(jnp.int32)
    tableT = table.T
    mesh = plsc.VectorSubcoreMesh(core_axis_name="c", subcore_axis_name="s")
    out_phys = pl.kernel(
        _emb_kernel,
        mesh=mesh,
        out_type=jax.ShapeDtypeStruct((S, DIM, NB), jnp.float32),
        scratch_types=(
            [pltpu.VMEM((NB,), jnp.int32) for _ in range(S_MAX)]
            + [pltpu.VMEM((NB,), jnp.float32) for _ in range(S_MAX)]
            + [
                pltpu.VMEM_SHARED((VOCAB,), jnp.float32),
                pltpu.SemaphoreType.DMA,
                pltpu.SemaphoreType.DMA,
                pltpu.SemaphoreType.DMA,
            ]
        ),
    )(xT, tableT)
    return out_phys.transpose(2, 0, 1)


# speedup vs baseline: 1.9846x; 1.0002x over previous
import math

import jax
import jax.numpy as jnp
from jax import lax
from jax.experimental import pallas as pl
from jax.experimental.pallas import tpu as pltpu
from jax.experimental.pallas import tpu_sc as plsc

DIM = 64
SCALE = 1.0 / math.sqrt(DIM)
LANES = 16
NUM_CORES = 2
NUM_SUBCORES = 16
VOCAB = 1000000
S = 200
NB = 4096
F_PER_CORE = DIM // NUM_CORES
S_MAX = 13
NRING = 3


def _emb_kernel(xT_hbm, tableT_hbm, out_hbm, *refs):
    idx_bufs = refs[:S_MAX]
    dat_bufs = refs[S_MAX:S_MAX + NRING]
    row_sp, sem_g, sem_w, sem_s = refs[S_MAX + NRING:]
    c = lax.axis_index("c")
    t = lax.axis_index("s")
    s0 = jnp.where(t < 8, t * 13, 104 + (t - 8) * 12)

    for k in range(S_MAX - 1):
        pltpu.sync_copy(xT_hbm.at[s0 + k], idx_bufs[k])

    @pl.when(t < 8)
    def _():
        pltpu.sync_copy(xT_hbm.at[s0 + (S_MAX - 1)], idx_bufs[S_MAX - 1])

    def stage_row(f):
        @pl.when(t == 0)
        def _():
            pltpu.async_copy(tableT_hbm.at[f], row_sp, sem_s)

    def wait_stage():
        @pl.when(t == 0)
        def _():
            pltpu.make_async_copy(tableT_hbm.at[0], row_sp, sem_s).wait()

    def scale_buf(buf):
        def scale_blk(v, c2):
            for u in range(8):
                sl = pl.ds((v * 8 + u) * LANES, LANES)
                buf[sl] = buf[sl] * SCALE
            return c2

        lax.fori_loop(0, NB // (8 * LANES), scale_blk, 0)

    stage_row(c * F_PER_CORE)
    wait_stage()
    plsc.subcore_barrier()

    def fire_gather(k):
        pltpu.async_copy(row_sp.at[idx_bufs[k]], dat_bufs[k % NRING], sem_g)

    def wait_gather(k):
        pltpu.make_async_copy(
            row_sp.at[idx_bufs[k]], dat_bufs[k % NRING], sem_g
        ).wait()

    def fire_write(k, f):
        pltpu.async_copy(dat_bufs[k % NRING], out_hbm.at[s0 + k, f], sem_w)

    def wait_write(k):
        pltpu.make_async_copy(
            dat_bufs[k % NRING], out_hbm.at[0, 0], sem_w
        ).wait()

    def feat_body(j, carry):
        f = c * F_PER_CORE + j

        fire_gather(0)
        fire_gather(1)

        for k in range(S_MAX):
            if k == 9:
                wait_gather(k)

                @pl.when(t < 8)
                def _():
                    scale_buf(dat_bufs[k % NRING])
                    fire_write(k, f)
            elif k in (10, 11):
                wait_gather(k)
            elif k == S_MAX - 1:
                @pl.when(t < 8)
                def _():
                    wait_gather(S_MAX - 1)
            else:
                wait_gather(k)
                scale_buf(dat_bufs[k % NRING])
                fire_write(k, f)

            nk = k + 2
            if nk == S_MAX - 1:
                @pl.when(t < 8)
                def _():
                    wait_write(k - 1)
                    fire_gather(S_MAX - 1)
            elif nk < S_MAX - 1:
                if k >= 1:
                    wait_write(k - 1)
                fire_gather(nk)

        plsc.subcore_barrier()

        @pl.when(j + 1 < F_PER_CORE)
        def _():
            stage_row(f + 1)

        @pl.when(t < 8)
        def _():
            for k in (10, 11, 12):
                scale_buf(dat_bufs[k % NRING])
                fire_write(k, f)

        @pl.when(t >= 8)
        def _():
            for k in (9, 10, 11):
                scale_buf(dat_bufs[k % NRING])
                fire_write(k, f)

        for k in range(NRING):
            wait_write(k)

        @pl.when(j + 1 < F_PER_CORE)
        def _():
            wait_stage()

        plsc.subcore_barrier()
        return carry

    lax.fori_loop(0, F_PER_CORE, feat_body, 0)


@jax.jit
def kernel(x, table):
    xT = x.T.astype(jnp.int32)
    tableT = table.T
    mesh = plsc.VectorSubcoreMesh(core_axis_name="c", subcore_axis_name="s")
    out_phys = pl.kernel(
        _emb_kernel,
        mesh=mesh,
        out_type=jax.ShapeDtypeStruct((S, DIM, NB), jnp.float32),
        scratch_types=(
            [pltpu.VMEM((NB,), jnp.int32) for _ in range(S_MAX)]
            + [pltpu.VMEM((NB,), jnp.float32) for _ in range(NRING)]
            + [
                pltpu.VMEM_SHARED((VOCAB,), jnp.float32),
                pltpu.SemaphoreType.DMA,
                pltpu.SemaphoreType.DMA,
                pltpu.SemaphoreType.DMA,
            ]
        ),
    )(xT, tableT)
    return out_phys.transpose(2, 0, 1)
